# baseline (speedup 1.0000x reference)
import functools

import jax
import jax.numpy as jnp
from jax import lax
from jax.experimental import pallas as pl
from jax.experimental.pallas import tpu as pltpu
from jax.experimental.pallas import tpu_sc as plsc

TM = 256
MAX_TILES = 24
NEG = -1


def _gate_body(x_ref, Wg_ref, bg_ref, Ws_ref, bs_ref,
               shared_ref, e0_ref, e1_ref, r0_ref, r1_ref, w0_ref, w1_ref,
               cnt_ref, base_ref, *, tm, n_e, n_blocks):
    i = pl.program_id(0)

    @pl.when(i == 0)
    def _():
        base_ref[...] = jnp.zeros_like(base_ref)

    x_blk = x_ref[...]
    s = jax.nn.sigmoid(
        jnp.dot(x_blk, Wg_ref[...], preferred_element_type=jnp.float32)
        + bg_ref[...])
    lane = jax.lax.broadcasted_iota(jnp.int32, s.shape, 1)
    v0 = jnp.max(s, axis=1, keepdims=True)
    i0 = jnp.argmax(s, axis=1, keepdims=True).astype(jnp.int32)
    s1 = jnp.where(lane == i0, -jnp.inf, s)
    v1 = jnp.max(s1, axis=1, keepdims=True)
    i1 = jnp.argmax(s1, axis=1, keepdims=True).astype(jnp.int32)
    denom = v0 + v1
    w0_ref[...] = v0 / denom
    w1_ref[...] = v1 / denom
    e0_ref[...] = i0
    e1_ref[...] = i1

    ohA = (lane == i0).astype(jnp.float32)
    ohB = (lane == i1).astype(jnp.float32)
    C = jnp.concatenate([ohA, ohB], axis=0)
    m = 2 * tm
    rio = jax.lax.broadcasted_iota(jnp.int32, (m, m), 0)
    cio = jax.lax.broadcasted_iota(jnp.int32, (m, m), 1)
    tri = (rio >= cio).astype(jnp.float32)
    R = jnp.dot(tri, C, preferred_element_type=jnp.float32)
    base = base_ref[...]
    ranks = jnp.sum(C * (R + base), axis=1, keepdims=True) - 1.0
    ranks = ranks.astype(jnp.int32)
    r0_ref[...] = ranks[:tm]
    r1_ref[...] = ranks[tm:]
    new_base = base + jnp.sum(C, axis=0, keepdims=True)
    base_ref[...] = new_base

    @pl.when(i == n_blocks - 1)
    def _():
        tmf = jnp.float32(tm)
        padded = jnp.ceil(new_base / tmf) * tmf
        r8 = jax.lax.broadcasted_iota(jnp.int32, (n_e, n_e), 0)
        c8 = jax.lax.broadcasted_iota(jnp.int32, (n_e, n_e), 1)
        upper = (r8 < c8).astype(jnp.float32)
        ps = jnp.dot(padded, upper, preferred_element_type=jnp.float32)
        cum = ps + padded
        cnt_ref[...] = jnp.concatenate([ps, cum], axis=1).astype(jnp.int32)

    shared_ref[...] = (
        jnp.dot(x_blk, Ws_ref[...], preferred_element_type=jnp.float32)
        + bs_ref[...])


def _gate(x, Wg, bg, Ws, bs):
    n, d = x.shape
    n_e = Wg.shape[1]
    tm = TM
    nb = n // tm
    body = functools.partial(_gate_body, tm=tm, n_e=n_e, n_blocks=nb)
    outs = pl.pallas_call(
        body,
        grid=(nb,),
        in_specs=[
            pl.BlockSpec((tm, d), lambda i: (i, 0)),
            pl.BlockSpec((d, n_e), lambda i: (0, 0)),
            pl.BlockSpec((1, n_e), lambda i: (0, 0)),
            pl.BlockSpec((d, d), lambda i: (0, 0)),
            pl.BlockSpec((1, d), lambda i: (0, 0)),
        ],
        out_specs=[
            pl.BlockSpec((tm, d), lambda i: (i, 0)),
            pl.BlockSpec((tm, 1), lambda i: (i, 0)),
            pl.BlockSpec((tm, 1), lambda i: (i, 0)),
            pl.BlockSpec((tm, 1), lambda i: (i, 0)),
            pl.BlockSpec((tm, 1), lambda i: (i, 0)),
            pl.BlockSpec((tm, 1), lambda i: (i, 0)),
            pl.BlockSpec((tm, 1), lambda i: (i, 0)),
            pl.BlockSpec((1, 16), lambda i: (0, 0)),
        ],
        out_shape=[
            jax.ShapeDtypeStruct((n, d), jnp.float32),
            jax.ShapeDtypeStruct((n, 1), jnp.int32),
            jax.ShapeDtypeStruct((n, 1), jnp.int32),
            jax.ShapeDtypeStruct((n, 1), jnp.int32),
            jax.ShapeDtypeStruct((n, 1), jnp.int32),
            jax.ShapeDtypeStruct((n, 1), jnp.float32),
            jax.ShapeDtypeStruct((n, 1), jnp.float32),
            jax.ShapeDtypeStruct((1, 16), jnp.int32),
        ],
        scratch_shapes=[pltpu.VMEM((1, n_e), jnp.float32)],
        compiler_params=pltpu.CompilerParams(
            dimension_semantics=("arbitrary",)),
    )(x, Wg, bg.reshape(1, n_e), Ws, bs.reshape(1, d))
    return outs


def _dispatch(x, e0r, e1r, r0r, r1r, meta16, n_pairs, p_tot):
    n, d = x.shape
    info = plsc.get_sparse_core_info()
    nw = info.num_cores * info.num_subcores
    chunk = n_pairs // nw
    mesh = plsc.VectorSubcoreMesh(core_axis_name="c", subcore_axis_name="s")

    @functools.partial(
        pl.kernel, mesh=mesh,
        out_type=[
            jax.ShapeDtypeStruct((p_tot, d), jnp.float32),
            jax.ShapeDtypeStruct((n_pairs,), jnp.int32),
        ],
        scratch_types=[
            pltpu.VMEM((chunk,), jnp.int32),
            pltpu.VMEM((chunk,), jnp.int32),
            pltpu.VMEM((chunk,), jnp.int32),
            pltpu.VMEM((chunk,), jnp.int32),
            pltpu.VMEM((chunk, d), jnp.float32),
            pltpu.VMEM((16,), jnp.int32),
            pltpu.SemaphoreType.DMA,
            pltpu.SemaphoreType.DMA,
        ],
        compiler_params=pltpu.CompilerParams(needs_layout_passes=False),
    )
    def k(x_hbm, e0_hbm, e1_hbm, r0_hbm, r1_hbm, meta_hbm, xs_hbm, pos_hbm,
          eid_v, rank_v, pos_v, tok_v, rows_v, meta_v,
          sem, sem2):
        wid = lax.axis_index("s") * info.num_cores + lax.axis_index("c")
        base = wid * chunk
        half = nw // 2

        @pl.when(wid < half)
        def _():
            pltpu.sync_copy(e0_hbm.at[pl.ds(base, chunk)], eid_v)
            pltpu.sync_copy(r0_hbm.at[pl.ds(base, chunk)], rank_v)

        @pl.when(wid >= half)
        def _():
            off = base - half * chunk
            pltpu.sync_copy(e1_hbm.at[pl.ds(off, chunk)], eid_v)
            pltpu.sync_copy(r1_hbm.at[pl.ds(off, chunk)], rank_v)

        pltpu.sync_copy(meta_hbm, meta_v)

        lane = lax.iota(jnp.int32, 16)
        for j in range(chunk // 16):
            ev = eid_v[pl.ds(16 * j, 16)]
            rv = rank_v[pl.ds(16 * j, 16)]
            psg = plsc.load_gather(meta_v, [ev])
            pos_v[pl.ds(16 * j, 16)] = psg + rv
            p = lane + (base + 16 * j)
            tok_v[pl.ds(16 * j, 16)] = p & (n - 1)

        pltpu.sync_copy(pos_v, pos_hbm.at[pl.ds(base, chunk)])
        pltpu.async_copy(x_hbm.at[tok_v], rows_v, sem).wait()
        pltpu.async_copy(rows_v, xs_hbm.at[pos_v], sem2).wait()

    return k(x, e0r, e1r, r0r, r1r, meta16)


def _ffn_body(te_ref, xs_ref, W1_ref, b1_ref, W2_ref, b2_ref, out_ref,
              acc_ref, *, nf, tm):
    f = pl.program_id(0)
    t = pl.program_id(1)
    e = te_ref[t]

    @pl.when(e >= 0)
    def _():
        h = jnp.dot(xs_ref[...], W1_ref[0],
                    preferred_element_type=jnp.float32) + b1_ref[0]
        h = jax.nn.gelu(h)
        contrib = jnp.dot(h, W2_ref[0], preferred_element_type=jnp.float32)
        row = pl.ds(t * tm, tm)

        @pl.when(f == 0)
        def _():
            acc_ref[row, :] = contrib

        @pl.when(f > 0)
        def _():
            acc_ref[row, :] += contrib

        @pl.when(f == nf - 1)
        def _():
            out_ref[...] = acc_ref[row, :] + b2_ref[0]


def _ffn(xs, te, W1, b1, W2, b2, n_tiles):
    _, d = xs.shape
    n_e, _, ff = W1.shape
    tf = min(768, ff)
    nf = ff // tf

    def w1map(f, t, te_ref):
        return (jnp.where(te_ref[t] < 0, n_e - 1, te_ref[t]), 0, f)

    def w2map(f, t, te_ref):
        return (jnp.where(te_ref[t] < 0, n_e - 1, te_ref[t]), f, 0)

    grid_spec = pltpu.PrefetchScalarGridSpec(
        num_scalar_prefetch=1,
        grid=(nf, n_tiles),
        in_specs=[
            pl.BlockSpec((TM, d), lambda f, t, te_ref: (t, 0)),
            pl.BlockSpec((1, d, tf), w1map),
            pl.BlockSpec((1, 1, tf), w1map),
            pl.BlockSpec((1, tf, d), w2map),
            pl.BlockSpec((1, 1, d), lambda f, t, te_ref: (
                jnp.where(te_ref[t] < 0, n_e - 1, te_ref[t]), 0, 0)),
        ],
        out_specs=pl.BlockSpec((TM, d), lambda f, t, te_ref: (t, 0)),
        scratch_shapes=[pltpu.VMEM((n_tiles * TM, d), jnp.float32)],
    )
    body = functools.partial(_ffn_body, nf=nf, tm=TM)
    return pl.pallas_call(
        body,
        grid_spec=grid_spec,
        out_shape=jax.ShapeDtypeStruct((n_tiles * TM, d), jnp.float32),
        compiler_params=pltpu.CompilerParams(
            dimension_semantics=("arbitrary", "arbitrary")),
    )(te, xs, W1, b1.reshape(n_e, 1, ff), W2, b2.reshape(n_e, 1, d))


def _gather2(ys, pos):
    n_pairs = pos.shape[0]
    n = n_pairs // 2
    d = ys.shape[1]
    info = plsc.get_sparse_core_info()
    nw = info.num_cores * info.num_subcores
    tchunk = n // nw
    mesh = plsc.VectorSubcoreMesh(core_axis_name="c", subcore_axis_name="s")

    @functools.partial(
        pl.kernel, mesh=mesh,
        out_type=[
            jax.ShapeDtypeStruct((n, d), jnp.float32),
            jax.ShapeDtypeStruct((n, d), jnp.float32),
        ],
        scratch_types=[
            pltpu.VMEM((tchunk,), jnp.int32),
            pltpu.VMEM((tchunk, d), jnp.float32),
            pltpu.SemaphoreType.DMA,
        ],
        compiler_params=pltpu.CompilerParams(needs_layout_passes=False),
    )
    def k(ys_hbm, pos_hbm, y0_hbm, y1_hbm, idx_v, rows_v, sem):
        wid = lax.axis_index("s") * info.num_cores + lax.axis_index("c")
        tb = wid * tchunk
        for pair_off, dst in ((0, y0_hbm), (n, y1_hbm)):
            pltpu.sync_copy(pos_hbm.at[pl.ds(pair_off + tb, tchunk)], idx_v)
            pltpu.async_copy(ys_hbm.at[idx_v], rows_v, sem).wait()
            pltpu.sync_copy(rows_v, dst.at[pl.ds(tb, tchunk)])

    return k(ys, pos)


def _wsum_body(y0_ref, y1_ref, w0_ref, w1_ref, sh_ref, out_ref):
    out_ref[...] = (w0_ref[...] * y0_ref[...] + w1_ref[...] * y1_ref[...]
                    + sh_ref[...])


def _wsum(y0, y1, w0, w1, shared):
    n, d = shared.shape
    tm = TM
    nb = n // tm
    return pl.pallas_call(
        _wsum_body,
        grid=(nb,),
        in_specs=[
            pl.BlockSpec((tm, d), lambda i: (i, 0)),
            pl.BlockSpec((tm, d), lambda i: (i, 0)),
            pl.BlockSpec((tm, 1), lambda i: (i, 0)),
            pl.BlockSpec((tm, 1), lambda i: (i, 0)),
            pl.BlockSpec((tm, d), lambda i: (i, 0)),
        ],
        out_specs=pl.BlockSpec((tm, d), lambda i: (i, 0)),
        out_shape=jax.ShapeDtypeStruct((n, d), jnp.float32),
        compiler_params=pltpu.CompilerParams(
            dimension_semantics=("arbitrary",)),
    )(y0, y1, w0, w1, shared)


def kernel(x, Wg, bg, W1, b1, W2, b2, Ws, bs):
    n, d = x.shape
    n_e, _, ff = W1.shape
    n_pairs = 2 * n
    p_tot = MAX_TILES * TM

    shared, e0, e1, r0, r1, w0, w1, counts = _gate(x, Wg, bg, Ws, bs)

    meta = counts[0]

    ps = meta[:n_e]
    total = meta[15]
    tiles = jnp.arange(MAX_TILES, dtype=jnp.int32) * TM
    acc = jnp.sum((ps[None, :] <= tiles[:, None]).astype(jnp.int32), axis=1)
    te = jnp.where(tiles < total, acc - 1, NEG).astype(jnp.int32)

    xs, pos = _dispatch(x, e0.reshape(n), e1.reshape(n),
                        r0.reshape(n), r1.reshape(n), meta, n_pairs, p_tot)
    ys = _ffn(xs, te, W1, b1, W2, b2, MAX_TILES)
    y0, y1 = _gather2(ys, pos)
    out = _wsum(y0, y1, w0, w1, shared)
    return out


# speedup vs baseline: 1.4127x; 1.4127x over previous
import functools

import jax
import jax.numpy as jnp
from jax import lax
from jax.experimental import pallas as pl
from jax.experimental.pallas import tpu as pltpu
from jax.experimental.pallas import tpu_sc as plsc

TM = 256
MAX_TILES = 24
NEG = -1


def _gate_body(x_ref, Wg_ref, bg_ref, Ws_ref, bs_ref,
               shared_ref, e0_ref, e1_ref, r0_ref, r1_ref, w0_ref, w1_ref,
               cnt_ref, base_ref, *, tm, n_e, n_blocks):
    i = pl.program_id(0)

    @pl.when(i == 0)
    def _():
        base_ref[...] = jnp.zeros_like(base_ref)

    x_blk = x_ref[...]
    s = jax.nn.sigmoid(
        jnp.dot(x_blk, Wg_ref[...], preferred_element_type=jnp.float32)
        + bg_ref[...])
    lane = jax.lax.broadcasted_iota(jnp.int32, s.shape, 1)
    v0 = jnp.max(s, axis=1, keepdims=True)
    i0 = jnp.argmax(s, axis=1, keepdims=True).astype(jnp.int32)
    s1 = jnp.where(lane == i0, -jnp.inf, s)
    v1 = jnp.max(s1, axis=1, keepdims=True)
    i1 = jnp.argmax(s1, axis=1, keepdims=True).astype(jnp.int32)
    denom = v0 + v1
    w0_ref[...] = v0 / denom
    w1_ref[...] = v1 / denom
    e0_ref[...] = i0
    e1_ref[...] = i1

    ohA = (lane == i0).astype(jnp.float32)
    ohB = (lane == i1).astype(jnp.float32)
    C = jnp.concatenate([ohA, ohB], axis=0)
    m = 2 * tm
    rio = jax.lax.broadcasted_iota(jnp.int32, (m, m), 0)
    cio = jax.lax.broadcasted_iota(jnp.int32, (m, m), 1)
    tri = (rio >= cio).astype(jnp.float32)
    R = jnp.dot(tri, C, preferred_element_type=jnp.float32)
    base = base_ref[...]
    ranks = jnp.sum(C * (R + base), axis=1, keepdims=True) - 1.0
    ranks = ranks.astype(jnp.int32)
    r0_ref[...] = ranks[:tm]
    r1_ref[...] = ranks[tm:]
    new_base = base + jnp.sum(C, axis=0, keepdims=True)
    base_ref[...] = new_base

    @pl.when(i == n_blocks - 1)
    def _():
        tmf = jnp.float32(tm)
        padded = jnp.ceil(new_base / tmf) * tmf
        r8 = jax.lax.broadcasted_iota(jnp.int32, (n_e, n_e), 0)
        c8 = jax.lax.broadcasted_iota(jnp.int32, (n_e, n_e), 1)
        upper = (r8 < c8).astype(jnp.float32)
        ps = jnp.dot(padded, upper, preferred_element_type=jnp.float32)
        cum = ps + padded
        cnt_ref[...] = jnp.concatenate([ps, cum], axis=1).astype(jnp.int32)

    shared_ref[...] = (
        jnp.dot(x_blk, Ws_ref[...], preferred_element_type=jnp.float32)
        + bs_ref[...])


def _gate(x, Wg, bg, Ws, bs):
    n, d = x.shape
    n_e = Wg.shape[1]
    tm = TM
    nb = n // tm
    body = functools.partial(_gate_body, tm=tm, n_e=n_e, n_blocks=nb)
    outs = pl.pallas_call(
        body,
        grid=(nb,),
        in_specs=[
            pl.BlockSpec((tm, d), lambda i: (i, 0)),
            pl.BlockSpec((d, n_e), lambda i: (0, 0)),
            pl.BlockSpec((1, n_e), lambda i: (0, 0)),
            pl.BlockSpec((d, d), lambda i: (0, 0)),
            pl.BlockSpec((1, d), lambda i: (0, 0)),
        ],
        out_specs=[
            pl.BlockSpec((tm, d), lambda i: (i, 0)),
            pl.BlockSpec((tm, 1), lambda i: (i, 0)),
            pl.BlockSpec((tm, 1), lambda i: (i, 0)),
            pl.BlockSpec((tm, 1), lambda i: (i, 0)),
            pl.BlockSpec((tm, 1), lambda i: (i, 0)),
            pl.BlockSpec((tm, 1), lambda i: (i, 0)),
            pl.BlockSpec((tm, 1), lambda i: (i, 0)),
            pl.BlockSpec((1, 16), lambda i: (0, 0)),
        ],
        out_shape=[
            jax.ShapeDtypeStruct((n, d), jnp.float32),
            jax.ShapeDtypeStruct((n, 1), jnp.int32),
            jax.ShapeDtypeStruct((n, 1), jnp.int32),
            jax.ShapeDtypeStruct((n, 1), jnp.int32),
            jax.ShapeDtypeStruct((n, 1), jnp.int32),
            jax.ShapeDtypeStruct((n, 1), jnp.float32),
            jax.ShapeDtypeStruct((n, 1), jnp.float32),
            jax.ShapeDtypeStruct((1, 16), jnp.int32),
        ],
        scratch_shapes=[pltpu.VMEM((1, n_e), jnp.float32)],
        compiler_params=pltpu.CompilerParams(
            dimension_semantics=("arbitrary",)),
    )(x, Wg, bg.reshape(1, n_e), Ws, bs.reshape(1, d))
    return outs


def _dispatch(x, e0r, e1r, r0r, r1r, meta16, n_pairs, p_tot):
    n, d = x.shape
    info = plsc.get_sparse_core_info()
    nw = info.num_cores * info.num_subcores
    chunk = n_pairs // nw
    mesh = plsc.VectorSubcoreMesh(core_axis_name="c", subcore_axis_name="s")

    @functools.partial(
        pl.kernel, mesh=mesh,
        out_type=[
            jax.ShapeDtypeStruct((p_tot, d), jnp.float32),
            jax.ShapeDtypeStruct((n_pairs,), jnp.int32),
        ],
        scratch_types=[
            pltpu.VMEM((chunk,), jnp.int32),
            pltpu.VMEM((chunk,), jnp.int32),
            pltpu.VMEM((chunk,), jnp.int32),
            pltpu.VMEM((chunk,), jnp.int32),
            pltpu.VMEM((chunk, d), jnp.float32),
            pltpu.VMEM((16,), jnp.int32),
            pltpu.SemaphoreType.DMA,
            pltpu.SemaphoreType.DMA,
        ],
        compiler_params=pltpu.CompilerParams(needs_layout_passes=False),
    )
    def k(x_hbm, e0_hbm, e1_hbm, r0_hbm, r1_hbm, meta_hbm, xs_hbm, pos_hbm,
          eid_v, rank_v, pos_v, tok_v, rows_v, meta_v,
          sem, sem2):
        wid = lax.axis_index("s") * info.num_cores + lax.axis_index("c")
        base = wid * chunk
        half = nw // 2

        @pl.when(wid < half)
        def _():
            pltpu.sync_copy(e0_hbm.at[pl.ds(base, chunk)], eid_v)
            pltpu.sync_copy(r0_hbm.at[pl.ds(base, chunk)], rank_v)

        @pl.when(wid >= half)
        def _():
            off = base - half * chunk
            pltpu.sync_copy(e1_hbm.at[pl.ds(off, chunk)], eid_v)
            pltpu.sync_copy(r1_hbm.at[pl.ds(off, chunk)], rank_v)

        pltpu.sync_copy(meta_hbm, meta_v)

        lane = lax.iota(jnp.int32, 16)
        for j in range(chunk // 16):
            ev = eid_v[pl.ds(16 * j, 16)]
            rv = rank_v[pl.ds(16 * j, 16)]
            psg = plsc.load_gather(meta_v, [ev])
            pos_v[pl.ds(16 * j, 16)] = psg + rv
            p = lane + (base + 16 * j)
            tok_v[pl.ds(16 * j, 16)] = p & (n - 1)

        pltpu.sync_copy(pos_v, pos_hbm.at[pl.ds(base, chunk)])
        pltpu.async_copy(x_hbm.at[tok_v], rows_v, sem).wait()
        pltpu.async_copy(rows_v, xs_hbm.at[pos_v], sem2).wait()

    return k(x, e0r, e1r, r0r, r1r, meta16)


def _ffn_body(te_ref, xs_ref, W1_ref, b1_ref, W2_ref, b2_ref, out_ref):
    t = pl.program_id(0)
    e = te_ref[t]

    @pl.when(e >= 0)
    def _():
        h = jnp.dot(xs_ref[...], W1_ref[0],
                    preferred_element_type=jnp.float32) + b1_ref[0]
        h = jax.nn.gelu(h)
        out_ref[...] = jnp.dot(h, W2_ref[0],
                               preferred_element_type=jnp.float32) + b2_ref[0]


def _ffn(xs, te, W1, b1, W2, b2, n_tiles):
    _, d = xs.shape
    n_e, _, ff = W1.shape

    def wmap(t, te_ref):
        return (jnp.where(te_ref[t] < 0, n_e - 1, te_ref[t]), 0, 0)

    grid_spec = pltpu.PrefetchScalarGridSpec(
        num_scalar_prefetch=1,
        grid=(n_tiles,),
        in_specs=[
            pl.BlockSpec((TM, d), lambda t, te_ref: (t, 0)),
            pl.BlockSpec((1, d, ff), wmap),
            pl.BlockSpec((1, 1, ff), wmap),
            pl.BlockSpec((1, ff, d), wmap),
            pl.BlockSpec((1, 1, d), wmap),
        ],
        out_specs=pl.BlockSpec((TM, d), lambda t, te_ref: (t, 0)),
    )
    return pl.pallas_call(
        _ffn_body,
        grid_spec=grid_spec,
        out_shape=jax.ShapeDtypeStruct((n_tiles * TM, d), jnp.float32),
        compiler_params=pltpu.CompilerParams(
            dimension_semantics=("arbitrary",)),
    )(te, xs, W1, b1.reshape(n_e, 1, ff), W2, b2.reshape(n_e, 1, d))


def _gather2(ys, pos):
    n_pairs = pos.shape[0]
    n = n_pairs // 2
    d = ys.shape[1]
    info = plsc.get_sparse_core_info()
    nw = info.num_cores * info.num_subcores
    tchunk = n // nw
    mesh = plsc.VectorSubcoreMesh(core_axis_name="c", subcore_axis_name="s")

    @functools.partial(
        pl.kernel, mesh=mesh,
        out_type=[
            jax.ShapeDtypeStruct((n, d), jnp.float32),
            jax.ShapeDtypeStruct((n, d), jnp.float32),
        ],
        scratch_types=[
            pltpu.VMEM((tchunk,), jnp.int32),
            pltpu.VMEM((tchunk, d), jnp.float32),
            pltpu.SemaphoreType.DMA,
        ],
        compiler_params=pltpu.CompilerParams(needs_layout_passes=False),
    )
    def k(ys_hbm, pos_hbm, y0_hbm, y1_hbm, idx_v, rows_v, sem):
        wid = lax.axis_index("s") * info.num_cores + lax.axis_index("c")
        tb = wid * tchunk
        for pair_off, dst in ((0, y0_hbm), (n, y1_hbm)):
            pltpu.sync_copy(pos_hbm.at[pl.ds(pair_off + tb, tchunk)], idx_v)
            pltpu.async_copy(ys_hbm.at[idx_v], rows_v, sem).wait()
            pltpu.sync_copy(rows_v, dst.at[pl.ds(tb, tchunk)])

    return k(ys, pos)


def _wsum_body(y0_ref, y1_ref, w0_ref, w1_ref, sh_ref, out_ref):
    out_ref[...] = (w0_ref[...] * y0_ref[...] + w1_ref[...] * y1_ref[...]
                    + sh_ref[...])


def _wsum(y0, y1, w0, w1, shared):
    n, d = shared.shape
    tm = TM
    nb = n // tm
    return pl.pallas_call(
        _wsum_body,
        grid=(nb,),
        in_specs=[
            pl.BlockSpec((tm, d), lambda i: (i, 0)),
            pl.BlockSpec((tm, d), lambda i: (i, 0)),
            pl.BlockSpec((tm, 1), lambda i: (i, 0)),
            pl.BlockSpec((tm, 1), lambda i: (i, 0)),
            pl.BlockSpec((tm, d), lambda i: (i, 0)),
        ],
        out_specs=pl.BlockSpec((tm, d), lambda i: (i, 0)),
        out_shape=jax.ShapeDtypeStruct((n, d), jnp.float32),
        compiler_params=pltpu.CompilerParams(
            dimension_semantics=("arbitrary",)),
    )(y0, y1, w0, w1, shared)


def kernel(x, Wg, bg, W1, b1, W2, b2, Ws, bs):
    n, d = x.shape
    n_e, _, ff = W1.shape
    n_pairs = 2 * n
    p_tot = MAX_TILES * TM

    shared, e0, e1, r0, r1, w0, w1, counts = _gate(x, Wg, bg, Ws, bs)

    meta = counts[0]

    ps = meta[:n_e]
    total = meta[15]
    tiles = jnp.arange(MAX_TILES, dtype=jnp.int32) * TM
    acc = jnp.sum((ps[None, :] <= tiles[:, None]).astype(jnp.int32), axis=1)
    te = jnp.where(tiles < total, acc - 1, NEG).astype(jnp.int32)

    xs, pos = _dispatch(x, e0.reshape(n), e1.reshape(n),
                        r0.reshape(n), r1.reshape(n), meta, n_pairs, p_tot)
    ys = _ffn(xs, te, W1, b1, W2, b2, MAX_TILES)
    y0, y1 = _gather2(ys, pos)
    out = _wsum(y0, y1, w0, w1, shared)
    return out


# speedup vs baseline: 1.4333x; 1.0146x over previous
import functools

import jax
import jax.numpy as jnp
from jax import lax
from jax.experimental import pallas as pl
from jax.experimental.pallas import tpu as pltpu
from jax.experimental.pallas import tpu_sc as plsc

TM = 256
MAX_TILES = 24
NEG = -1


def _gate_body(x_ref, Wg_ref, bg_ref,
               e0_ref, e1_ref, r0_ref, r1_ref, w0_ref, w1_ref,
               cnt_ref, te_ref, base_ref, *, tm, n_e, n_blocks, max_tiles):
    i = pl.program_id(0)

    @pl.when(i == 0)
    def _():
        base_ref[...] = jnp.zeros_like(base_ref)

    x_blk = x_ref[...]
    s = jax.nn.sigmoid(
        jnp.dot(x_blk, Wg_ref[...], preferred_element_type=jnp.float32)
        + bg_ref[...])
    lane = jax.lax.broadcasted_iota(jnp.int32, s.shape, 1)
    v0 = jnp.max(s, axis=1, keepdims=True)
    i0 = jnp.argmax(s, axis=1, keepdims=True).astype(jnp.int32)
    s1 = jnp.where(lane == i0, -jnp.inf, s)
    v1 = jnp.max(s1, axis=1, keepdims=True)
    i1 = jnp.argmax(s1, axis=1, keepdims=True).astype(jnp.int32)
    denom = v0 + v1
    w0_ref[...] = v0 / denom
    w1_ref[...] = v1 / denom
    e0_ref[...] = i0
    e1_ref[...] = i1

    ohA = (lane == i0).astype(jnp.float32)
    ohB = (lane == i1).astype(jnp.float32)
    C = jnp.concatenate([ohA, ohB], axis=0)
    m = 2 * tm
    rio = jax.lax.broadcasted_iota(jnp.int32, (m, m), 0)
    cio = jax.lax.broadcasted_iota(jnp.int32, (m, m), 1)
    tri = (rio >= cio).astype(jnp.float32)
    R = jnp.dot(tri, C, preferred_element_type=jnp.float32)
    base = base_ref[...]
    ranks = jnp.sum(C * (R + base), axis=1, keepdims=True) - 1.0
    ranks = ranks.astype(jnp.int32)
    r0_ref[...] = ranks[:tm]
    r1_ref[...] = ranks[tm:]
    new_base = base + jnp.sum(C, axis=0, keepdims=True)
    base_ref[...] = new_base

    @pl.when(i == n_blocks - 1)
    def _():
        tmf = jnp.float32(tm)
        padded = jnp.ceil(new_base / tmf) * tmf
        r8 = jax.lax.broadcasted_iota(jnp.int32, (n_e, n_e), 0)
        c8 = jax.lax.broadcasted_iota(jnp.int32, (n_e, n_e), 1)
        upper = (r8 < c8).astype(jnp.float32)
        ps = jnp.dot(padded, upper, preferred_element_type=jnp.float32)
        cum = ps + padded
        cnt_ref[...] = jnp.concatenate([ps, cum], axis=1).astype(jnp.int32)

        eye = (r8 == c8).astype(jnp.float32)
        padded_c = lax.dot_general(
            eye, padded, (((1,), (1,)), ((), ())),
            preferred_element_type=jnp.float32)
        lower = (r8 > c8).astype(jnp.float32)
        ps_c = jnp.dot(lower, padded_c,
                       preferred_element_type=jnp.float32)
        tiles = jax.lax.broadcasted_iota(
            jnp.int32, (1, max_tiles), 1).astype(jnp.float32) * tmf
        hit = (ps_c <= tiles).astype(jnp.float32)
        acc = jnp.sum(hit, axis=0, keepdims=True)
        total = jnp.sum(padded, axis=1, keepdims=True)
        tev = jnp.where(tiles < total, acc - 1.0, -1.0)
        te_ref[...] = tev.astype(jnp.int32)


def _gate(x, Wg, bg, max_tiles):
    n, d = x.shape
    n_e = Wg.shape[1]
    tm = TM
    nb = n // tm
    body = functools.partial(_gate_body, tm=tm, n_e=n_e, n_blocks=nb,
                             max_tiles=max_tiles)
    outs = pl.pallas_call(
        body,
        grid=(nb,),
        in_specs=[
            pl.BlockSpec((tm, d), lambda i: (i, 0)),
            pl.BlockSpec((d, n_e), lambda i: (0, 0)),
            pl.BlockSpec((1, n_e), lambda i: (0, 0)),
        ],
        out_specs=[
            pl.BlockSpec((tm, 1), lambda i: (i, 0)),
            pl.BlockSpec((tm, 1), lambda i: (i, 0)),
            pl.BlockSpec((tm, 1), lambda i: (i, 0)),
            pl.BlockSpec((tm, 1), lambda i: (i, 0)),
            pl.BlockSpec((tm, 1), lambda i: (i, 0)),
            pl.BlockSpec((tm, 1), lambda i: (i, 0)),
            pl.BlockSpec((1, 16), lambda i: (0, 0)),
            pl.BlockSpec((1, max_tiles), lambda i: (0, 0)),
        ],
        out_shape=[
            jax.ShapeDtypeStruct((n, 1), jnp.int32),
            jax.ShapeDtypeStruct((n, 1), jnp.int32),
            jax.ShapeDtypeStruct((n, 1), jnp.int32),
            jax.ShapeDtypeStruct((n, 1), jnp.int32),
            jax.ShapeDtypeStruct((n, 1), jnp.float32),
            jax.ShapeDtypeStruct((n, 1), jnp.float32),
            jax.ShapeDtypeStruct((1, 16), jnp.int32),
            jax.ShapeDtypeStruct((1, max_tiles), jnp.int32),
        ],
        scratch_shapes=[pltpu.VMEM((1, n_e), jnp.float32)],
        compiler_params=pltpu.CompilerParams(
            dimension_semantics=("arbitrary",)),
    )(x, Wg, bg.reshape(1, n_e))
    return outs


def _dispatch(x, e0r, e1r, r0r, r1r, meta16, n_pairs, p_tot):
    n, d = x.shape
    info = plsc.get_sparse_core_info()
    nw = info.num_cores * info.num_subcores
    chunk = n_pairs // nw
    mesh = plsc.VectorSubcoreMesh(core_axis_name="c", subcore_axis_name="s")

    @functools.partial(
        pl.kernel, mesh=mesh,
        out_type=[
            jax.ShapeDtypeStruct((p_tot, d), jnp.float32),
            jax.ShapeDtypeStruct((n_pairs,), jnp.int32),
        ],
        scratch_types=[
            pltpu.VMEM((chunk,), jnp.int32),
            pltpu.VMEM((chunk,), jnp.int32),
            pltpu.VMEM((chunk,), jnp.int32),
            pltpu.VMEM((chunk,), jnp.int32),
            pltpu.VMEM((chunk, d), jnp.float32),
            pltpu.VMEM((16,), jnp.int32),
            pltpu.SemaphoreType.DMA,
            pltpu.SemaphoreType.DMA,
        ],
        compiler_params=pltpu.CompilerParams(needs_layout_passes=False),
    )
    def k(x_hbm, e0_hbm, e1_hbm, r0_hbm, r1_hbm, meta_hbm, xs_hbm, pos_hbm,
          eid_v, rank_v, pos_v, tok_v, rows_v, meta_v,
          sem, sem2):
        wid = lax.axis_index("s") * info.num_cores + lax.axis_index("c")
        base = wid * chunk
        half = nw // 2

        @pl.when(wid < half)
        def _():
            pltpu.sync_copy(e0_hbm.at[pl.ds(base, chunk)], eid_v)
            pltpu.sync_copy(r0_hbm.at[pl.ds(base, chunk)], rank_v)

        @pl.when(wid >= half)
        def _():
            off = base - half * chunk
            pltpu.sync_copy(e1_hbm.at[pl.ds(off, chunk)], eid_v)
            pltpu.sync_copy(r1_hbm.at[pl.ds(off, chunk)], rank_v)

        pltpu.sync_copy(meta_hbm, meta_v)

        lane = lax.iota(jnp.int32, 16)
        for j in range(chunk // 16):
            ev = eid_v[pl.ds(16 * j, 16)]
            rv = rank_v[pl.ds(16 * j, 16)]
            psg = plsc.load_gather(meta_v, [ev])
            pos_v[pl.ds(16 * j, 16)] = psg + rv
            p = lane + (base + 16 * j)
            tok_v[pl.ds(16 * j, 16)] = p & (n - 1)

        pltpu.sync_copy(pos_v, pos_hbm.at[pl.ds(base, chunk)])
        pltpu.async_copy(x_hbm.at[tok_v], rows_v, sem).wait()
        pltpu.async_copy(rows_v, xs_hbm.at[pos_v], sem2).wait()

    return k(x, e0r, e1r, r0r, r1r, meta16)


def _ffn_body(te_ref, xs_ref, W1_ref, b1_ref, W2_ref, b2_ref, out_ref):
    t = pl.program_id(0)
    e = te_ref[0, t]

    @pl.when(e >= 0)
    def _():
        h = jnp.dot(xs_ref[...], W1_ref[0],
                    preferred_element_type=jnp.float32) + b1_ref[0]
        h = jax.nn.gelu(h)
        out_ref[...] = jnp.dot(h, W2_ref[0],
                               preferred_element_type=jnp.float32) + b2_ref[0]


def _ffn(xs, te, W1, b1, W2, b2, n_tiles):
    _, d = xs.shape
    n_e, _, ff = W1.shape

    def wmap(t, te_ref):
        return (jnp.where(te_ref[0, t] < 0, n_e - 1, te_ref[0, t]), 0, 0)

    grid_spec = pltpu.PrefetchScalarGridSpec(
        num_scalar_prefetch=1,
        grid=(n_tiles,),
        in_specs=[
            pl.BlockSpec((TM, d), lambda t, te_ref: (t, 0)),
            pl.BlockSpec((1, d, ff), wmap),
            pl.BlockSpec((1, 1, ff), wmap),
            pl.BlockSpec((1, ff, d), wmap),
            pl.BlockSpec((1, 1, d), wmap),
        ],
        out_specs=pl.BlockSpec((TM, d), lambda t, te_ref: (t, 0)),
    )
    return pl.pallas_call(
        _ffn_body,
        grid_spec=grid_spec,
        out_shape=jax.ShapeDtypeStruct((n_tiles * TM, d), jnp.float32),
        compiler_params=pltpu.CompilerParams(
            dimension_semantics=("arbitrary",)),
    )(te, xs, W1, b1.reshape(n_e, 1, ff), W2, b2.reshape(n_e, 1, d))


def _gather2(ys, pos):
    n_pairs = pos.shape[0]
    n = n_pairs // 2
    d = ys.shape[1]
    info = plsc.get_sparse_core_info()
    nw = info.num_cores * info.num_subcores
    tchunk = n // nw
    mesh = plsc.VectorSubcoreMesh(core_axis_name="c", subcore_axis_name="s")

    @functools.partial(
        pl.kernel, mesh=mesh,
        out_type=[
            jax.ShapeDtypeStruct((n, d), jnp.float32),
            jax.ShapeDtypeStruct((n, d), jnp.float32),
        ],
        scratch_types=[
            pltpu.VMEM((tchunk,), jnp.int32),
            pltpu.VMEM((tchunk, d), jnp.float32),
            pltpu.SemaphoreType.DMA,
        ],
        compiler_params=pltpu.CompilerParams(needs_layout_passes=False),
    )
    def k(ys_hbm, pos_hbm, y0_hbm, y1_hbm, idx_v, rows_v, sem):
        wid = lax.axis_index("s") * info.num_cores + lax.axis_index("c")
        tb = wid * tchunk
        for pair_off, dst in ((0, y0_hbm), (n, y1_hbm)):
            pltpu.sync_copy(pos_hbm.at[pl.ds(pair_off + tb, tchunk)], idx_v)
            pltpu.async_copy(ys_hbm.at[idx_v], rows_v, sem).wait()
            pltpu.sync_copy(rows_v, dst.at[pl.ds(tb, tchunk)])

    return k(ys, pos)


def _wsum_body(y0_ref, y1_ref, w0_ref, w1_ref, x_ref, Ws_ref, bs_ref,
               out_ref):
    shared = (jnp.dot(x_ref[...], Ws_ref[...],
                      preferred_element_type=jnp.float32) + bs_ref[...])
    out_ref[...] = (w0_ref[...] * y0_ref[...] + w1_ref[...] * y1_ref[...]
                    + shared)


def _wsum(y0, y1, w0, w1, x, Ws, bs):
    n, d = x.shape
    tm = TM
    nb = n // tm
    return pl.pallas_call(
        _wsum_body,
        grid=(nb,),
        in_specs=[
            pl.BlockSpec((tm, d), lambda i: (i, 0)),
            pl.BlockSpec((tm, d), lambda i: (i, 0)),
            pl.BlockSpec((tm, 1), lambda i: (i, 0)),
            pl.BlockSpec((tm, 1), lambda i: (i, 0)),
            pl.BlockSpec((tm, d), lambda i: (i, 0)),
            pl.BlockSpec((d, d), lambda i: (0, 0)),
            pl.BlockSpec((1, d), lambda i: (0, 0)),
        ],
        out_specs=pl.BlockSpec((tm, d), lambda i: (i, 0)),
        out_shape=jax.ShapeDtypeStruct((n, d), jnp.float32),
        compiler_params=pltpu.CompilerParams(
            dimension_semantics=("arbitrary",)),
    )(y0, y1, w0, w1, x, Ws, bs.reshape(1, d))


def kernel(x, Wg, bg, W1, b1, W2, b2, Ws, bs):
    n, d = x.shape
    n_e, _, ff = W1.shape
    n_pairs = 2 * n
    p_tot = MAX_TILES * TM

    e0, e1, r0, r1, w0, w1, counts, te = _gate(x, Wg, bg, MAX_TILES)

    meta = counts[0]
    xs, pos = _dispatch(x, e0.reshape(n), e1.reshape(n),
                        r0.reshape(n), r1.reshape(n), meta, n_pairs, p_tot)
    ys = _ffn(xs, te, W1, b1, W2, b2, MAX_TILES)
    y0, y1 = _gather2(ys, pos)
    out = _wsum(y0, y1, w0, w1, x, Ws, bs)
    return out
